# baseline (speedup 1.0000x reference)
import functools

import jax
import jax.numpy as jnp
from jax.experimental import pallas as pl
from jax.experimental.pallas import tpu as pltpu

BM = 400


def _layer1_body(adj_ref, xfull_ref, xi_ref, w0_ref, w1_ref, b_ref, h_ref):
    y = jnp.dot(
        adj_ref[...].astype(jnp.bfloat16),
        xfull_ref[...].astype(jnp.bfloat16),
        preferred_element_type=jnp.float32,
    )
    h = (
        jnp.dot(xi_ref[...], w0_ref[...], preferred_element_type=jnp.float32)
        + jnp.dot(y, w1_ref[...], preferred_element_type=jnp.float32)
        + b_ref[...]
    )
    h_ref[...] = jnp.maximum(h, 0.0)


def _layer2_body(adj_ref, hfull_ref, hi_ref, w0_ref, w1_ref, b_ref, o_ref):
    z = jnp.dot(
        adj_ref[...].astype(jnp.bfloat16),
        hfull_ref[...].astype(jnp.bfloat16),
        preferred_element_type=jnp.float32,
    )
    o = (
        jnp.dot(hi_ref[...], w0_ref[...], preferred_element_type=jnp.float32)
        + jnp.dot(z, w1_ref[...], preferred_element_type=jnp.float32)
        + b_ref[...]
    )
    m = jnp.max(o, axis=1, keepdims=True)
    e = jnp.exp(o - m)
    lse = jnp.log(jnp.sum(e, axis=1, keepdims=True))
    o_ref[...] = o - m - lse


def _cheb_layer(body, a_feat, adj, feat_dim_out, w0, w1, b):
    n, f_in = a_feat.shape
    grid = (n // BM,)
    return pl.pallas_call(
        body,
        grid=grid,
        in_specs=[
            pl.BlockSpec((BM, n), lambda i: (i, 0)),
            pl.BlockSpec((n, f_in), lambda i: (0, 0)),
            pl.BlockSpec((BM, f_in), lambda i: (i, 0)),
            pl.BlockSpec((f_in, feat_dim_out), lambda i: (0, 0)),
            pl.BlockSpec((f_in, feat_dim_out), lambda i: (0, 0)),
            pl.BlockSpec((1, feat_dim_out), lambda i: (0, 0)),
        ],
        out_specs=pl.BlockSpec((BM, feat_dim_out), lambda i: (i, 0)),
        out_shape=jax.ShapeDtypeStruct((n, feat_dim_out), jnp.float32),
    )(adj, a_feat, a_feat, w0, w1, b)


@jax.jit
def kernel(x, adj, W1_0, W1_1, b1, W2_0, W2_1, b2):
    hid = W1_0.shape[1]
    c_out = W2_0.shape[1]
    h = _cheb_layer(_layer1_body, x, adj, hid, W1_0, W1_1, b1.reshape(1, hid))
    return _cheb_layer(_layer2_body, h, adj, c_out, W2_0, W2_1, b2.reshape(1, c_out))


# speedup vs baseline: 1.0642x; 1.0642x over previous
import jax
import jax.numpy as jnp
from jax.experimental import pallas as pl
from jax.experimental.pallas import tpu as pltpu

BM = 400


def _body(adj_ref, x_ref, w10_ref, w11_ref, b1_ref, w20_ref, w21_ref, b2_ref,
          o_ref, h_ref):
    p = pl.program_id(0)
    i = pl.program_id(1)
    rows = pl.ds(i * BM, BM)

    @pl.when(p == 0)
    def _():
        y = jnp.dot(adj_ref[...], x_ref[...], preferred_element_type=jnp.float32)
        h = (
            jnp.dot(x_ref[rows, :], w10_ref[...], preferred_element_type=jnp.float32)
            + jnp.dot(y, w11_ref[...], preferred_element_type=jnp.float32)
            + b1_ref[...]
        )
        h_ref[rows, :] = jnp.maximum(h, 0.0)

    @pl.when(p == 1)
    def _():
        z = jnp.dot(adj_ref[...], h_ref[...], preferred_element_type=jnp.float32)
        o = (
            jnp.dot(h_ref[rows, :], w20_ref[...], preferred_element_type=jnp.float32)
            + jnp.dot(z, w21_ref[...], preferred_element_type=jnp.float32)
            + b2_ref[...]
        )
        m = jnp.max(o, axis=1, keepdims=True)
        e = jnp.exp(o - m)
        lse = jnp.log(jnp.sum(e, axis=1, keepdims=True))
        o_ref[...] = o - m - lse


@jax.jit
def kernel(x, adj, W1_0, W1_1, b1, W2_0, W2_1, b2):
    n, f_in = x.shape
    hid = W1_0.shape[1]
    c_out = W2_0.shape[1]
    grid = (2, n // BM)
    return pl.pallas_call(
        _body,
        grid=grid,
        in_specs=[
            pl.BlockSpec((BM, n), lambda p, i: (i, 0)),
            pl.BlockSpec((n, f_in), lambda p, i: (0, 0)),
            pl.BlockSpec((f_in, hid), lambda p, i: (0, 0)),
            pl.BlockSpec((f_in, hid), lambda p, i: (0, 0)),
            pl.BlockSpec((1, hid), lambda p, i: (0, 0)),
            pl.BlockSpec((hid, c_out), lambda p, i: (0, 0)),
            pl.BlockSpec((hid, c_out), lambda p, i: (0, 0)),
            pl.BlockSpec((1, c_out), lambda p, i: (0, 0)),
        ],
        out_specs=pl.BlockSpec((BM, c_out), lambda p, i: (i, 0)),
        out_shape=jax.ShapeDtypeStruct((n, c_out), jnp.float32),
        scratch_shapes=[pltpu.VMEM((n, hid), jnp.float32)],
    )(adj, x, W1_0, W1_1, b1.reshape(1, hid), W2_0, W2_1, b2.reshape(1, c_out))
